# Initial kernel scaffold; baseline (speedup 1.0000x reference)
import jax
import jax.numpy as jnp
from jax.experimental import pallas as pl


def kernel(input_ids, attention_mask, wte):
    raise NotImplementedError("write your pallas kernel here")


# speedup vs baseline: 1.6227x; 1.6227x over previous
import functools

import jax
import jax.numpy as jnp
from jax import lax
from jax.experimental import pallas as pl
from jax.experimental.pallas import tpu as pltpu
from jax.experimental.pallas import tpu_sc as plsc

NC = 2
NS = 16
NW = NC * NS

VOCAB = 100000
D = 1024
B_FLAT = 4 * 8192
B_PER_W = B_FLAT // NW
CHUNK = 64
NCHUNK = B_PER_W // CHUNK


def _gather_body(table_hbm, idx_hbm, out_hbm, idx_v, rows_v, gsem):
    wid = lax.axis_index("s") * NC + lax.axis_index("c")
    base = wid * B_PER_W
    pltpu.sync_copy(idx_hbm.at[pl.ds(base, B_PER_W)], idx_v)

    def step(i, _):
        idx_c = idx_v.at[pl.ds(i * CHUNK, CHUNK)]
        pltpu.async_copy(table_hbm.at[idx_c], rows_v, gsem).wait()
        pltpu.sync_copy(rows_v, out_hbm.at[pl.ds(base + i * CHUNK, CHUNK)])
        return _

    lax.fori_loop(0, NCHUNK, step, 0)


@functools.partial(jax.jit, static_argnames=())
def _sc_gather(wte, flat_ids):
    mesh = plsc.VectorSubcoreMesh(
        core_axis_name="c", subcore_axis_name="s",
        num_cores=NC, num_subcores=NS,
    )
    return pl.kernel(
        _gather_body,
        out_type=jax.ShapeDtypeStruct((B_FLAT, D), jnp.float32),
        mesh=mesh,
        scratch_types=[
            pltpu.VMEM((B_PER_W,), jnp.int32),
            pltpu.VMEM((CHUNK, D), jnp.float32),
            pltpu.SemaphoreType.DMA,
        ],
    )(wte, flat_ids)


def _mask_body(am_ref, out_ref):
    m = am_ref[...].astype(jnp.bfloat16)
    out_ref[...] = (1.0 - m) * jnp.finfo(jnp.bfloat16).min


def _mask_expand(attention_mask):
    b, s = attention_mask.shape
    return pl.pallas_call(
        _mask_body,
        out_shape=jax.ShapeDtypeStruct((b, s), jnp.bfloat16),
    )(attention_mask)


def kernel(input_ids, attention_mask, wte):
    input_shape = input_ids.shape
    flat_ids = input_ids.reshape(-1)
    hidden = _sc_gather(wte, flat_ids)
    hidden = hidden.reshape(*input_shape, D)
    batch = input_ids.reshape(-1, input_shape[-1]).shape[0]
    am = _mask_expand(attention_mask.reshape(batch, -1))
    am = am[:, None, None, :]
    return (hidden, am)


# speedup vs baseline: 1.7192x; 1.0594x over previous
import functools

import jax
import jax.numpy as jnp
from jax import lax
from jax.experimental import pallas as pl
from jax.experimental.pallas import tpu as pltpu
from jax.experimental.pallas import tpu_sc as plsc

NC = 2
NS = 16
NW = NC * NS

VOCAB = 100000
D = 1024
B_FLAT = 4 * 8192
B_PER_W = B_FLAT // NW
CHUNK = 32
NCHUNK = B_PER_W // CHUNK


def _gather_body(table_hbm, idx_hbm, out_hbm, idx_v, rows0, rows1,
                 gsem0, gsem1, osem0, osem1):
    wid = lax.axis_index("s") * NC + lax.axis_index("c")
    base = wid * B_PER_W
    pltpu.sync_copy(idx_hbm.at[pl.ds(base, B_PER_W)], idx_v)

    rows = (rows0, rows1)
    gsem = (gsem0, gsem1)
    osem = (osem0, osem1)

    def idx_c(i):
        return idx_v.at[pl.ds(i * CHUNK, CHUNK)]

    def out_c(i):
        return out_hbm.at[pl.ds(base + i * CHUNK, CHUNK)]

    for i in (0, 1):
        pltpu.async_copy(table_hbm.at[idx_c(i)], rows[i], gsem[i]).wait()
        pltpu.async_copy(rows[i], out_c(i), osem[i])

    @pl.loop(2, NCHUNK, step=2)
    def _middle(iv):
        for b in (0, 1):
            i = iv + b
            pltpu.make_async_copy(rows[b], out_c(i - 2), osem[b]).wait()
            pltpu.async_copy(table_hbm.at[idx_c(i)], rows[b], gsem[b]).wait()
            pltpu.async_copy(rows[b], out_c(i), osem[b])

    for b in (0, 1):
        pltpu.make_async_copy(rows[b], out_c(NCHUNK - 2 + b), osem[b]).wait()


@functools.partial(jax.jit, static_argnames=())
def _sc_gather(wte, flat_ids):
    mesh = plsc.VectorSubcoreMesh(
        core_axis_name="c", subcore_axis_name="s",
        num_cores=NC, num_subcores=NS,
    )
    return pl.kernel(
        _gather_body,
        out_type=jax.ShapeDtypeStruct((B_FLAT, D), jnp.float32),
        mesh=mesh,
        scratch_types=[
            pltpu.VMEM((B_PER_W,), jnp.int32),
            pltpu.VMEM((CHUNK, D), jnp.float32),
            pltpu.VMEM((CHUNK, D), jnp.float32),
            pltpu.SemaphoreType.DMA,
            pltpu.SemaphoreType.DMA,
            pltpu.SemaphoreType.DMA,
            pltpu.SemaphoreType.DMA,
        ],
    )(wte, flat_ids)


def _mask_body(am_ref, out_ref):
    m = am_ref[...].astype(jnp.bfloat16)
    out_ref[...] = (1.0 - m) * jnp.finfo(jnp.bfloat16).min


def _mask_expand(attention_mask):
    b, s = attention_mask.shape
    return pl.pallas_call(
        _mask_body,
        out_shape=jax.ShapeDtypeStruct((b, s), jnp.bfloat16),
    )(attention_mask)


def kernel(input_ids, attention_mask, wte):
    input_shape = input_ids.shape
    flat_ids = input_ids.reshape(-1)
    hidden = _sc_gather(wte, flat_ids)
    hidden = hidden.reshape(*input_shape, D)
    batch = input_ids.reshape(-1, input_shape[-1]).shape[0]
    am = _mask_expand(attention_mask.reshape(batch, -1))
    am = am[:, None, None, :]
    return (hidden, am)
